# Initial kernel scaffold; baseline (speedup 1.0000x reference)
import jax
import jax.numpy as jnp
from jax.experimental import pallas as pl


def kernel(bias, L):
    raise NotImplementedError("write your pallas kernel here")


# speedup vs baseline: 1.8029x; 1.8029x over previous
import functools
import math

import jax
import jax.numpy as jnp
from jax import lax
from jax.experimental import pallas as pl
from jax.experimental.pallas import tpu as pltpu
from jax.experimental.pallas import tpu_sc as plsc

_L = 4096
_NB = 32
_MAXD = 128
_TBL = 2 * _L
_NS = 8
_NWORKERS = 32
_ROWS_PER_W = _L // _NWORKERS
_GROUPS = _ROWS_PER_W // _NS


def _table_body(bias_ref, t_ref):
    s = lax.broadcasted_iota(jnp.int32, (_NS, _TBL), 0)
    m = lax.broadcasted_iota(jnp.int32, (_NS, _TBL), 1)
    rel = jnp.abs(s + m - (_L - 1))
    half = _NB // 2
    small = rel < half
    log_rel = jnp.log(rel.astype(jnp.float32) / half + 1e-06)
    log_max = math.log(max(_MAXD / half, 1.0))
    scaled = log_rel / (log_max + 1e-06) * (_NB - half - 1)
    large = (half + jnp.clip(jnp.floor(scaled), 0, _NB - half - 1)).astype(
        jnp.int32)
    bucket = jnp.where(small, rel, large)
    acc = jnp.zeros((_NS, _TBL), jnp.float32)
    for b in range(_NB):
        acc = acc + jnp.where(bucket == b, bias_ref[b], 0.0)
    t_ref[...] = acc


def _fanout_body(t_hbm, out_hbm, t_v, sem):
    wid = lax.axis_index("s") * 2 + lax.axis_index("c")
    pltpu.sync_copy(t_hbm, t_v)
    row0 = wid * _ROWS_PER_W

    def group(g, carry):
        q8 = (_L - _NS) - row0 - g * _NS
        copies = [
            pltpu.async_copy(
                t_v.at[pl.ds(pl.multiple_of((_NS - 1 - b) * _TBL + q8, 8),
                             _L)],
                out_hbm.at[pl.ds(pl.multiple_of((row0 + g * _NS + b) * _L, 8),
                                 _L)],
                sem,
            )
            for b in range(_NS)
        ]
        for c in copies:
            c.wait()
        return carry

    lax.fori_loop(0, _GROUPS, group, 0)


@jax.jit
def _impl(bias):
    t = pl.pallas_call(
        _table_body,
        in_specs=[pl.BlockSpec(memory_space=pltpu.SMEM)],
        out_specs=pl.BlockSpec(memory_space=pltpu.VMEM),
        out_shape=jax.ShapeDtypeStruct((_NS, _TBL), jnp.float32),
    )(bias)
    fanout = pl.kernel(
        _fanout_body,
        out_type=jax.ShapeDtypeStruct((_L * _L,), jnp.float32),
        mesh=plsc.VectorSubcoreMesh(core_axis_name="c", subcore_axis_name="s"),
        scratch_types=[
            pltpu.VMEM((_NS * _TBL,), jnp.float32),
            pltpu.SemaphoreType.DMA,
        ],
    )
    return fanout(t.reshape(-1)).reshape(_L, _L)


def kernel(bias, L):
    return _impl(bias)


# speedup vs baseline: 3.6402x; 2.0190x over previous
import functools
import math

import jax
import jax.numpy as jnp
from jax import lax
from jax.experimental import pallas as pl
from jax.experimental.pallas import tpu as pltpu
from jax.experimental.pallas import tpu_sc as plsc

_L = 4096
_NB = 32
_MAXD = 128
_TBL = 2 * _L
_NROWS = 128
_NTEC = 32
_TR_PER_TEC = 16


def _u_body(bias_ref, u_ref):
    row = lax.broadcasted_iota(jnp.int32, (_NROWS, _TBL), 0)
    m = lax.broadcasted_iota(jnp.int32, (_NROWS, _TBL), 1)
    shift = 8 * (row >> 3) + 7 - (row & 7)
    rel = jnp.abs(shift + m - (_L - 1))
    half = _NB // 2
    small = rel < half
    log_rel = jnp.log(rel.astype(jnp.float32) / half + 1e-06)
    log_max = math.log(max(_MAXD / half, 1.0))
    scaled = log_rel / (log_max + 1e-06) * (_NB - half - 1)
    large = (half + jnp.clip(jnp.floor(scaled), 0, _NB - half - 1)).astype(
        jnp.int32)
    bucket = jnp.where(small, rel, large)
    acc = jnp.zeros((_NROWS, _TBL), jnp.float32)
    for b in range(_NB):
        acc = acc + jnp.where(bucket == b, bias_ref[b], 0.0)
    u_ref[...] = acc


def _fanout_body(u_hbm, out_hbm, u_v, sem):
    wid = lax.axis_index("s") * 2 + lax.axis_index("c")
    rho = wid >> 1
    u0 = (wid & 1) * _TR_PER_TEC
    pltpu.sync_copy(u_hbm.at[pl.ds(pl.multiple_of(8 * rho, 8), 8), :], u_v)

    copies = []
    for v in range(_TR_PER_TEC):
        u = u0 + v
        tr = 15 - rho + 16 * u
        k0 = 31 - u
        copies.append(
            pltpu.async_copy(
                u_v.at[:, pl.ds(pl.multiple_of(128 * k0, 128), _L)],
                out_hbm.at[pl.ds(pl.multiple_of(8 * tr, 8), 8), :],
                sem,
            )
        )
    for c in copies:
        c.wait()


@jax.jit
def _impl(bias):
    u = pl.pallas_call(
        _u_body,
        in_specs=[pl.BlockSpec(memory_space=pltpu.SMEM)],
        out_specs=pl.BlockSpec(memory_space=pltpu.VMEM),
        out_shape=jax.ShapeDtypeStruct((_NROWS, _TBL), jnp.float32),
    )(bias)
    fanout = pl.kernel(
        _fanout_body,
        out_type=jax.ShapeDtypeStruct((_L, _L), jnp.float32),
        mesh=plsc.VectorSubcoreMesh(core_axis_name="c", subcore_axis_name="s"),
        scratch_types=[
            pltpu.VMEM((8, _TBL), jnp.float32),
            pltpu.SemaphoreType.DMA,
        ],
    )
    return fanout(u)


def kernel(bias, L):
    return _impl(bias)


# speedup vs baseline: 4.7445x; 1.3033x over previous
import functools
import math

import jax
import jax.numpy as jnp
from jax import lax
from jax.experimental import pallas as pl
from jax.experimental.pallas import tpu as pltpu
from jax.experimental.pallas import tpu_sc as plsc

_L = 4096
_NB = 32
_MAXD = 128
_TBL = 2 * _L
_NROWS = 128
_NTEC = 32
_TR_PER_TEC = 16


def _u_body(bias_ref, u_ref):
    w = _TBL + 2 * _NROWS
    b = lax.broadcasted_iota(jnp.int32, (8, w), 0)
    m = lax.broadcasted_iota(jnp.int32, (8, w), 1)
    rel = jnp.abs((7 - b) + m - (_L - 1))
    half = _NB // 2
    small = rel < half
    log_rel = jnp.log(rel.astype(jnp.float32) / half + 1e-06)
    log_max = math.log(max(_MAXD / half, 1.0))
    scaled = log_rel / (log_max + 1e-06) * (_NB - half - 1)
    large = (half + jnp.clip(jnp.floor(scaled), 0, _NB - half - 1)).astype(
        jnp.int32)
    bucket = jnp.where(small, rel, large)
    acc = jnp.zeros((8, w), jnp.float32)
    for k in range(_NB):
        acc = acc + jnp.where(bucket == k, bias_ref[k], 0.0)
    for rho in range(16):
        u_ref[8 * rho:8 * rho + 8, :] = lax.slice(
            acc, (0, 8 * rho), (8, 8 * rho + _TBL))


def _fanout_body(u_hbm, out_hbm, u_v, sem):
    wid = lax.axis_index("s") * 2 + lax.axis_index("c")
    rho = wid >> 1
    u0 = (wid & 1) * _TR_PER_TEC
    pltpu.sync_copy(u_hbm.at[pl.ds(pl.multiple_of(8 * rho, 8), 8), :], u_v)

    copies = []
    for v in range(_TR_PER_TEC):
        u = u0 + v
        tr = 15 - rho + 16 * u
        k0 = 31 - u
        copies.append(
            pltpu.async_copy(
                u_v.at[:, pl.ds(pl.multiple_of(128 * k0, 128), _L)],
                out_hbm.at[pl.ds(pl.multiple_of(8 * tr, 8), 8), :],
                sem,
            )
        )
    for c in copies:
        c.wait()


@jax.jit
def _impl(bias):
    u = pl.pallas_call(
        _u_body,
        in_specs=[pl.BlockSpec(memory_space=pltpu.SMEM)],
        out_specs=pl.BlockSpec(memory_space=pltpu.VMEM),
        out_shape=jax.ShapeDtypeStruct((_NROWS, _TBL), jnp.float32),
    )(bias)
    fanout = pl.kernel(
        _fanout_body,
        out_type=jax.ShapeDtypeStruct((_L, _L), jnp.float32),
        mesh=plsc.VectorSubcoreMesh(core_axis_name="c", subcore_axis_name="s"),
        scratch_types=[
            pltpu.VMEM((8, _TBL), jnp.float32),
            pltpu.SemaphoreType.DMA,
        ],
    )
    return fanout(u)


def kernel(bias, L):
    return _impl(bias)


# speedup vs baseline: 4.8627x; 1.0249x over previous
import functools
import math

import jax
import jax.numpy as jnp
from jax import lax
from jax.experimental import pallas as pl
from jax.experimental.pallas import tpu as pltpu
from jax.experimental.pallas import tpu_sc as plsc

_L = 4096
_NB = 32
_MAXD = 128
_TBL = 2 * _L
_NROWS = 128
_NTEC = 32
_TR_PER_TEC = 16


def _u_body(bias_ref, u_ref):
    w = _TBL + 2 * _NROWS
    b = lax.broadcasted_iota(jnp.int32, (8, w), 0)
    m = lax.broadcasted_iota(jnp.int32, (8, w), 1)
    rel = jnp.abs((7 - b) + m - (_L - 1))
    half = _NB // 2
    small = rel < half
    log_rel = jnp.log(rel.astype(jnp.float32) / half + 1e-06)
    log_max = math.log(max(_MAXD / half, 1.0))
    scaled = log_rel / (log_max + 1e-06) * (_NB - half - 1)
    large = (half + jnp.clip(jnp.floor(scaled), 0, _NB - half - 1)).astype(
        jnp.int32)
    bucket = jnp.where(small, rel, large)
    acc = jnp.zeros((8, w), jnp.float32)
    for k in range(_NB):
        acc = acc + jnp.where(bucket == k, bias_ref[k], 0.0)
    for rho in range(16):
        u_ref[8 * rho:8 * rho + 8, :] = lax.slice(
            acc, (0, 8 * rho), (8, 8 * rho + _TBL))


def _fanout_body(u_hbm, out_hbm, u_v, sem_in, sem_out):
    wid = lax.axis_index("s") * 2 + lax.axis_index("c")
    rho = wid >> 1
    half = wid & 1
    u0 = half * _TR_PER_TEC
    rows = u_hbm.at[pl.ds(pl.multiple_of(8 * rho, 8), 8), :]
    lane0 = pl.multiple_of(2048 - half * 2048, 2048)
    w1 = pltpu.async_copy(rows.at[:, pl.ds(lane0, _L)],
                          u_v.at[:, pl.ds(lane0, _L)], sem_in)
    w2 = pltpu.async_copy(rows.at[:, pl.ds(lane0 + _L, 2048)],
                          u_v.at[:, pl.ds(lane0 + _L, 2048)], sem_in)
    w1.wait()

    copies = []
    for v in range(_TR_PER_TEC):
        u = u0 + (_TR_PER_TEC - 1 - v)
        tr = 15 - rho + 16 * u
        k0 = 31 - u
        copies.append(
            pltpu.async_copy(
                u_v.at[:, pl.ds(pl.multiple_of(128 * k0, 128), _L)],
                out_hbm.at[pl.ds(pl.multiple_of(8 * tr, 8), 8), :],
                sem_out,
            )
        )
        if v == 0:
            w2.wait()
    for c in copies:
        c.wait()


@jax.jit
def _impl(bias):
    u = pl.pallas_call(
        _u_body,
        in_specs=[pl.BlockSpec(memory_space=pltpu.SMEM)],
        out_specs=pl.BlockSpec(memory_space=pltpu.VMEM),
        out_shape=jax.ShapeDtypeStruct((_NROWS, _TBL), jnp.float32),
    )(bias)
    fanout = pl.kernel(
        _fanout_body,
        out_type=jax.ShapeDtypeStruct((_L, _L), jnp.float32),
        mesh=plsc.VectorSubcoreMesh(core_axis_name="c", subcore_axis_name="s"),
        scratch_types=[
            pltpu.VMEM((8, _TBL), jnp.float32),
            pltpu.SemaphoreType.DMA,
            pltpu.SemaphoreType.DMA,
        ],
    )
    return fanout(u)


def kernel(bias, L):
    return _impl(bias)


# speedup vs baseline: 4.9025x; 1.0082x over previous
import functools
import math

import jax
import jax.numpy as jnp
from jax import lax
from jax.experimental import pallas as pl
from jax.experimental.pallas import tpu as pltpu
from jax.experimental.pallas import tpu_sc as plsc

_L = 4096
_NB = 32
_MAXD = 128
_TBL = 2 * _L
_NROWS = 128
_NTEC = 32
_TR_PER_TEC = 16


def _u_body(bias_ref, u_ref):
    w = _TBL + 2 * _NROWS
    p0, pw = 3952, 384
    b = lax.broadcasted_iota(jnp.int32, (8, pw), 0)
    m = lax.broadcasted_iota(jnp.int32, (8, pw), 1)
    rel = jnp.abs((7 - b) + (m + p0) - (_L - 1))
    half = _NB // 2
    small = rel < half
    log_rel = jnp.log(rel.astype(jnp.float32) / half + 1e-06)
    log_max = math.log(max(_MAXD / half, 1.0))
    scaled = log_rel / (log_max + 1e-06) * (_NB - half - 1)
    large = (half + jnp.clip(jnp.floor(scaled), 0, _NB - half - 1)).astype(
        jnp.int32)
    bucket = jnp.where(small, rel, large)
    patch = jnp.zeros((8, pw), jnp.float32)
    for k in range(_NB):
        patch = patch + jnp.where(bucket == k, bias_ref[k], 0.0)
    c31 = jnp.full((8, p0), bias_ref[_NB - 1], jnp.float32)
    c31r = jnp.full((8, w - p0 - pw), bias_ref[_NB - 1], jnp.float32)
    acc = jnp.concatenate([c31, patch, c31r], axis=1)
    for rho in range(16):
        u_ref[8 * rho:8 * rho + 8, :] = lax.slice(
            acc, (0, 8 * rho), (8, 8 * rho + _TBL))


def _fanout_body(u_hbm, out_hbm, u_v, sem_in, sem_out):
    wid = lax.axis_index("s") * 2 + lax.axis_index("c")
    rho = wid >> 1
    half = wid & 1
    u0 = half * _TR_PER_TEC
    rows = u_hbm.at[pl.ds(pl.multiple_of(8 * rho, 8), 8), :]
    lane0 = pl.multiple_of(2048 - half * 2048, 2048)
    w1 = pltpu.async_copy(rows.at[:, pl.ds(lane0, _L)],
                          u_v.at[:, pl.ds(lane0, _L)], sem_in)
    w2 = pltpu.async_copy(rows.at[:, pl.ds(lane0 + _L, 2048)],
                          u_v.at[:, pl.ds(lane0 + _L, 2048)], sem_in)
    w1.wait()

    copies = []
    for v in range(_TR_PER_TEC):
        u = u0 + (_TR_PER_TEC - 1 - v)
        tr = 15 - rho + 16 * u
        k0 = 31 - u
        copies.append(
            pltpu.async_copy(
                u_v.at[:, pl.ds(pl.multiple_of(128 * k0, 128), _L)],
                out_hbm.at[pl.ds(pl.multiple_of(8 * tr, 8), 8), :],
                sem_out,
            )
        )
        if v == 0:
            w2.wait()
    for c in copies:
        c.wait()


@jax.jit
def _impl(bias):
    u = pl.pallas_call(
        _u_body,
        in_specs=[pl.BlockSpec(memory_space=pltpu.SMEM)],
        out_specs=pl.BlockSpec(memory_space=pltpu.VMEM),
        out_shape=jax.ShapeDtypeStruct((_NROWS, _TBL), jnp.float32),
    )(bias)
    fanout = pl.kernel(
        _fanout_body,
        out_type=jax.ShapeDtypeStruct((_L, _L), jnp.float32),
        mesh=plsc.VectorSubcoreMesh(core_axis_name="c", subcore_axis_name="s"),
        scratch_types=[
            pltpu.VMEM((8, _TBL), jnp.float32),
            pltpu.SemaphoreType.DMA,
            pltpu.SemaphoreType.DMA,
        ],
    )
    return fanout(u)


def kernel(bias, L):
    return _impl(bias)


# speedup vs baseline: 4.9262x; 1.0048x over previous
import math

import jax
import jax.numpy as jnp
from jax import lax
from jax.experimental import pallas as pl
from jax.experimental.pallas import tpu as pltpu
from jax.experimental.pallas import tpu_sc as plsc

_L = 4096
_NB = 32
_MAXD = 128
_TBL = 2 * _L
_NROWS = 128
_TR_PER_TEC = 16


def _u_body(bias_ref, u_ref):
    w = _TBL + 2 * _NROWS
    p0, pw = 3952, 384
    b = lax.broadcasted_iota(jnp.int32, (8, pw), 0)
    m = lax.broadcasted_iota(jnp.int32, (8, pw), 1)
    rel = jnp.abs((7 - b) + (m + p0) - (_L - 1))
    half = _NB // 2
    small = rel < half
    log_rel = jnp.log(rel.astype(jnp.float32) / half + 1e-06)
    log_max = math.log(max(_MAXD / half, 1.0))
    scaled = log_rel / (log_max + 1e-06) * (_NB - half - 1)
    large = (half + jnp.clip(jnp.floor(scaled), 0, _NB - half - 1)).astype(
        jnp.int32)
    bucket = jnp.where(small, rel, large)
    patch = jnp.zeros((8, pw), jnp.float32)
    for k in range(_NB):
        patch = patch + jnp.where(bucket == k, bias_ref[k], 0.0)
    c31 = jnp.full((8, p0), bias_ref[_NB - 1], jnp.float32)
    c31r = jnp.full((8, w - p0 - pw), bias_ref[_NB - 1], jnp.float32)
    acc = jnp.concatenate([c31, patch, c31r], axis=1)
    for rho in range(16):
        u_ref[8 * rho:8 * rho + 8, :] = lax.slice(
            acc, (0, 8 * rho), (8, 8 * rho + _TBL))


def _fanout_body(u_hbm, out_hbm, u_v, sem_in, sem_out):
    wid = lax.axis_index("s") * 2 + lax.axis_index("c")
    rho = wid >> 1
    half = wid & 1
    u0 = half * _TR_PER_TEC
    rows = u_hbm.at[pl.ds(pl.multiple_of(8 * rho, 8), 8), :]
    lane0 = pl.multiple_of(2048 - half * 2048, 2048)
    w1 = pltpu.async_copy(rows.at[:, pl.ds(lane0, _L)],
                          u_v.at[:, pl.ds(lane0, _L)], sem_in)
    w2 = pltpu.async_copy(rows.at[:, pl.ds(lane0 + _L, 2048)],
                          u_v.at[:, pl.ds(lane0 + _L, 2048)], sem_in)
    w1.wait()

    copies = []
    for v in range(_TR_PER_TEC):
        u = u0 + (_TR_PER_TEC - 1 - v)
        tr = 15 - rho + 16 * u
        k0 = 31 - u
        copies.append(
            pltpu.async_copy(
                u_v.at[:, pl.ds(pl.multiple_of(128 * k0, 128), _L)],
                out_hbm.at[pl.ds(pl.multiple_of(8 * tr, 8), 8), :],
                sem_out,
            )
        )
        if v == 0:
            w2.wait()
    for c in copies:
        c.wait()


@jax.jit
def _impl(bias):
    u = pl.pallas_call(
        _u_body,
        in_specs=[pl.BlockSpec(memory_space=pltpu.SMEM)],
        out_specs=pl.BlockSpec(memory_space=pltpu.VMEM),
        out_shape=jax.ShapeDtypeStruct((_NROWS, _TBL), jnp.float32),
    )(bias)
    fanout = pl.kernel(
        _fanout_body,
        out_type=jax.ShapeDtypeStruct((_L, _L), jnp.float32),
        mesh=plsc.VectorSubcoreMesh(core_axis_name="c", subcore_axis_name="s"),
        scratch_types=[
            pltpu.VMEM((8, _TBL), jnp.float32),
            pltpu.SemaphoreType.DMA,
            pltpu.SemaphoreType.DMA,
        ],
    )
    return fanout(u)


def kernel(bias, L):
    return _impl(bias)
